# Initial kernel scaffold; baseline (speedup 1.0000x reference)
import jax
import jax.numpy as jnp
from jax.experimental import pallas as pl


def kernel(x, edge_index, batch_size, params):
    raise NotImplementedError("write your pallas kernel here")


# speedup vs baseline: 8.7769x; 8.7769x over previous
import functools

import jax
import jax.numpy as jnp
from jax import lax
from jax.experimental import pallas as pl
from jax.experimental.pallas import tpu as pltpu
import jax.experimental.pallas.tpu_sc as plsc

N = 10000
E = 320000
H = 128
EPS = 0.1
BN_EPS = 1e-5

NC = 2
NS = 16
NW = NC * NS
COPY_TILES = 10
NPC = N // COPY_TILES

G2 = 128
RPT2 = 80
E2 = NW * RPT2 * G2
EPT2 = E2 // NW
BLK = 8
NBLK = RPT2 // BLK
NPAD = N + 16

_mesh = plsc.VectorSubcoreMesh(core_axis_name="c", subcore_axis_name="s")


def _wid(c, s):
    return c * NS + s


HALVES = 2
NH = N // HALVES


def _deg_body(dstf, out, dst_v, hist_v, tot_v):
    c = lax.axis_index("c")
    s = lax.axis_index("s")
    t = _wid(c, s)
    pltpu.sync_copy(dstf.at[pl.ds(t * EPT2, EPT2)], dst_v)
    lane = jnp.arange(16, dtype=jnp.int32)
    zeros = jnp.zeros((16,), jnp.float32)
    ones = jnp.ones((16,), jnp.float32)

    for p in range(HALVES):
        def zstep(i, carry):
            hist_v[pl.ds(i * 16, 16)] = zeros
            return carry

        lax.fori_loop(0, NH, zstep, 0)

        base = p * NH

        def hstep(g, carry):
            dv = dst_v[pl.ds(g * 16, 16)]
            local = dv - base
            m = (local >= 0) & (local < NH)
            lidx = jnp.clip(local, 0, NH - 1) * 16 + lane
            plsc.addupdate_scatter(hist_v, [lidx], ones, mask=m)
            return carry

        lax.fori_loop(0, EPT2 // 16, hstep, 0)

        def rstep(r, carry):
            row16 = jnp.full((16,), r * 16, jnp.int32) + lane
            acc = zeros
            for i in range(16):
                acc = acc + plsc.load_gather(
                    hist_v, [row16 * 16 + jnp.full((16,), i, jnp.int32)])
            tot_v[pl.ds(r * 16, 16)] = acc
            return carry

        lax.fori_loop(0, NH // 16, rstep, 0)
        pltpu.sync_copy(tot_v, out.at[pl.ds(t * N + base, NH)])


_sc_params = pltpu.CompilerParams(needs_layout_passes=False)

_deg_kernel = pl.kernel(
    _deg_body,
    out_type=jax.ShapeDtypeStruct((NW * N,), jnp.float32),
    mesh=_mesh,
    compiler_params=_sc_params,
    scratch_types=[
        pltpu.VMEM((EPT2,), jnp.int32),
        pltpu.VMEM((NH * 16,), jnp.float32),
        pltpu.VMEM((NH,), jnp.float32),
    ],
)


def _w_body(hl, hr, dis, srcf, dstf, wout, hl_v, hr_v, dis_v, srcf_v,
            dstf_v, w_v):
    c = lax.axis_index("c")
    s = lax.axis_index("s")
    t = _wid(c, s)
    base = t * EPT2
    pltpu.sync_copy(hl, hl_v)
    pltpu.sync_copy(hr, hr_v)
    pltpu.sync_copy(dis, dis_v)
    pltpu.sync_copy(srcf.at[pl.ds(base, EPT2)], srcf_v)
    pltpu.sync_copy(dstf.at[pl.ds(base, EPT2)], dstf_v)
    lane = jnp.arange(16, dtype=jnp.int32)

    def wstep(g, carry):
        sl = pl.ds(g * 16, 16)
        sv = srcf_v[sl]
        dv = dstf_v[sl]
        a = plsc.load_gather(hl_v, [sv]) + plsc.load_gather(hr_v, [dv])
        ex = jnp.exp(-2.0 * jnp.abs(a))
        th = jnp.sign(a) * ((1.0 - ex) / (1.0 + ex))
        w = th * plsc.load_gather(dis_v, [sv]) * plsc.load_gather(dis_v, [dv])
        eidx = jnp.full((16,), base + g * 16, jnp.int32) + lane
        w_v[sl] = jnp.where(eidx < E, w, 0.0)
        return carry

    lax.fori_loop(0, EPT2 // 16, wstep, 0)
    pltpu.sync_copy(w_v, wout.at[pl.ds(base, EPT2)])


_w_kernel = pl.kernel(
    _w_body,
    out_type=jax.ShapeDtypeStruct((E2,), jnp.float32),
    mesh=_mesh,
    compiler_params=_sc_params,
    scratch_types=[
        pltpu.VMEM((N,), jnp.float32),
        pltpu.VMEM((N,), jnp.float32),
        pltpu.VMEM((N,), jnp.float32),
        pltpu.VMEM((EPT2,), jnp.int32),
        pltpu.VMEM((EPT2,), jnp.int32),
        pltpu.VMEM((EPT2,), jnp.float32),
    ],
)


def _scat_body(h, w, src2d, dst2d, zerosH, out,
               src_v, dst_v, w_v, rows_v, acc_sh, sem):
    c = lax.axis_index("c")
    s = lax.axis_index("s")
    t = _wid(c, s)
    lane = jnp.arange(16, dtype=jnp.int32)

    @pl.when(s == 0)
    def _():
        pltpu.sync_copy(zerosH, acc_sh)

    plsc.subcore_barrier()

    def blk(b, carry):
        row0 = t * RPT2 + b * BLK
        pltpu.sync_copy(src2d.at[pl.ds(row0, BLK)], src_v)
        pltpu.sync_copy(dst2d.at[pl.ds(row0, BLK)], dst_v)
        pltpu.sync_copy(w.at[pl.ds(t * EPT2 + b * (BLK * G2), BLK * G2)], w_v)
        for jj in range(BLK):
            pltpu.async_copy(h.at[src_v.at[jj]], rows_v, sem).wait()

            def scale(r, carry2):
                wv = plsc.load_gather(
                    w_v, [jnp.full((16,), jj * G2 + r, jnp.int32)])
                rfull = jnp.full((16,), r, jnp.int32)
                for k in range(H // 16):
                    col = jnp.full((16,), k * 16, jnp.int32) + lane
                    v = plsc.load_gather(rows_v, [rfull, col])
                    plsc.store_scatter(rows_v, [rfull, col], v * wv)
                return carry2

            lax.fori_loop(0, G2, scale, 0)
            pltpu.sync_copy(rows_v, acc_sh.at[dst_v.at[jj]], add=True)
        return carry

    lax.fori_loop(0, NBLK, blk, 0)
    plsc.subcore_barrier()

    @pl.when(s < COPY_TILES)
    def _():
        pltpu.sync_copy(acc_sh.at[pl.ds(s * NPC, NPC)],
                        out.at[pl.ds(c * N + s * NPC, NPC)])


_scat_kernel = pl.kernel(
    _scat_body,
    out_type=jax.ShapeDtypeStruct((2 * N, H), jnp.float32),
    mesh=_mesh,
    compiler_params=_sc_params,
    scratch_types=[
        pltpu.VMEM((BLK, G2), jnp.int32),
        pltpu.VMEM((BLK, G2), jnp.int32),
        pltpu.VMEM((BLK * G2,), jnp.float32),
        pltpu.VMEM((G2, H), jnp.float32),
        pltpu.VMEM_SHARED((NPAD, H), jnp.float32),
        pltpu.SemaphoreType.DMA,
    ],
)


def _bn(z, g, b, rm, rv):
    return (z - rm) / jnp.sqrt(rv + BN_EPS) * g + b


def _pre_body(x_ref, w_ref, b_ref, al_ref, ar_ref, dp_ref,
              x0_ref, hl_ref, hr_ref, dis_ref, ws_ref):
    x0 = jnp.maximum(
        jnp.dot(x_ref[...], w_ref[...], preferred_element_type=jnp.float32)
        + b_ref[...][None, :], 0.0)
    x0_ref[...] = x0
    hl = jnp.dot(x0, al_ref[...], preferred_element_type=jnp.float32)
    hr = jnp.dot(x0, ar_ref[...], preferred_element_type=jnp.float32)
    hl_ref[...] = hl
    hr_ref[...] = hr
    deg = jnp.sum(dp_ref[...], axis=0)[:, None] + 1.0
    dis = lax.rsqrt(deg)
    dis_ref[...] = dis
    ws_ref[...] = jnp.tanh(hl + hr) * dis * dis


def _tc_pre(x, w, b, al, ar, degpart):
    return pl.pallas_call(
        _pre_body,
        out_shape=(
            jax.ShapeDtypeStruct((N, H), jnp.float32),
            jax.ShapeDtypeStruct((N, 1), jnp.float32),
            jax.ShapeDtypeStruct((N, 1), jnp.float32),
            jax.ShapeDtypeStruct((N, 1), jnp.float32),
            jax.ShapeDtypeStruct((N, 1), jnp.float32),
        ),
    )(x, w, b, al, ar, degpart)


def _mid_body(part_ref, hprev_ref, x0_ref, ws_ref, dis_ref,
              g_ref, b_ref, rm_ref, rv_ref, al_ref, ar_ref,
              h1_ref, hl_ref, hr_ref, ws1_ref):
    part = part_ref[...]
    out = (part[0:N] + part[N:2 * N]
           + ws_ref[...] * hprev_ref[...] + EPS * x0_ref[...])
    h1 = _bn(jnp.maximum(out, 0.0), g_ref[...][None, :], b_ref[...][None, :],
             rm_ref[...][None, :], rv_ref[...][None, :])
    h1_ref[...] = h1
    hl = jnp.dot(h1, al_ref[...], preferred_element_type=jnp.float32)
    hr = jnp.dot(h1, ar_ref[...], preferred_element_type=jnp.float32)
    hl_ref[...] = hl
    hr_ref[...] = hr
    dis = dis_ref[...]
    ws1_ref[...] = jnp.tanh(hl + hr) * dis * dis


def _tc_mid(part, hprev, x0, ws, dis, g, b, rm, rv, al, ar):
    return pl.pallas_call(
        _mid_body,
        out_shape=(
            jax.ShapeDtypeStruct((N, H), jnp.float32),
            jax.ShapeDtypeStruct((N, 1), jnp.float32),
            jax.ShapeDtypeStruct((N, 1), jnp.float32),
            jax.ShapeDtypeStruct((N, 1), jnp.float32),
        ),
    )(part, hprev, x0, ws, dis, g, b, rm, rv, al, ar)


def _post_body(part_ref, hprev_ref, x0_ref, ws_ref,
               g_ref, b_ref, rm_ref, rv_ref,
               w1_ref, b1_ref, g1_ref, bb1_ref, rm1_ref, rv1_ref,
               w2_ref, b2_ref, g2_ref, bb2_ref, rm2_ref, rv2_ref,
               w3_ref, b3_ref, bs_ref, z_ref, h2_ref):
    part = part_ref[...]
    out = (part[0:N] + part[N:2 * N]
           + ws_ref[...] * hprev_ref[...] + EPS * x0_ref[...])
    h2_ref[...] = _bn(jnp.maximum(out, 0.0), g_ref[...][None, :],
                      b_ref[...][None, :], rm_ref[...][None, :],
                      rv_ref[...][None, :])
    start = bs_ref[0] - 1024
    z = h2_ref[pl.ds(start, 1024), :]
    z = jnp.maximum(_bn(
        jnp.dot(z, w1_ref[...], preferred_element_type=jnp.float32)
        + b1_ref[...][None, :],
        g1_ref[...][None, :], bb1_ref[...][None, :],
        rm1_ref[...][None, :], rv1_ref[...][None, :]), 0.0)
    z = jnp.maximum(_bn(
        jnp.dot(z, w2_ref[...], preferred_element_type=jnp.float32)
        + b2_ref[...][None, :],
        g2_ref[...][None, :], bb2_ref[...][None, :],
        rm2_ref[...][None, :], rv2_ref[...][None, :]), 0.0)
    z_ref[...] = (jnp.dot(z, w3_ref[...], preferred_element_type=jnp.float32)
                  + b3_ref[...][None, :])


def _tc_post(part, hprev, x0, ws, g, b, rm, rv, p, bs):
    in_specs = [pl.BlockSpec(memory_space=pltpu.VMEM) for _ in range(22)]
    in_specs.append(pl.BlockSpec(memory_space=pltpu.SMEM))
    return pl.pallas_call(
        _post_body,
        out_shape=jax.ShapeDtypeStruct((1024, 1), jnp.float32),
        in_specs=in_specs,
        out_specs=pl.BlockSpec(memory_space=pltpu.VMEM),
        scratch_shapes=[pltpu.VMEM((N, H), jnp.float32)],
    )(part, hprev, x0, ws, g, b, rm, rv,
      p['W1'], p['b1'], p['bn1_g'], p['bn1_b'], p['bn1_rm'], p['bn1_rv'],
      p['W2'], p['b2'], p['bn2_g'], p['bn2_b'], p['bn2_rm'], p['bn2_rv'],
      p['W3'], p['b3'], bs)


def kernel(x, edge_index, batch_size, params):
    src = edge_index[0]
    dst = edge_index[1]
    srcp = jnp.concatenate([src, jnp.zeros((E2 - E,), jnp.int32)])
    dstp = jnp.concatenate([dst, jnp.full((E2 - E,), N, jnp.int32)])
    src2d = srcp.reshape(NW * RPT2, G2)
    dst2d = dstp.reshape(NW * RPT2, G2)
    zerosH = jnp.zeros((NPAD, H), jnp.float32)

    degpart = _deg_kernel(dstp).reshape(NW, N)

    c0, c1 = params['convs'][0], params['convs'][1]
    al0 = c0['a_l'].reshape(H, 1)
    ar0 = c0['a_r'].reshape(H, 1)
    al1 = c1['a_l'].reshape(H, 1)
    ar1 = c1['a_r'].reshape(H, 1)

    x0, hl0, hr0, dis, ws0 = _tc_pre(x, params['W_in'], params['b_in'],
                                     al0, ar0, degpart)

    w1 = _w_kernel(hl0.reshape(N), hr0.reshape(N), dis.reshape(N), srcp, dstp)
    part1 = _scat_kernel(x0, w1, src2d, dst2d, zerosH)

    h1, hl1, hr1, ws1 = _tc_mid(part1, x0, x0, ws0, dis,
                                c0['bn_g'], c0['bn_b'], c0['bn_rm'],
                                c0['bn_rv'], al1, ar1)

    w2 = _w_kernel(hl1.reshape(N), hr1.reshape(N), dis.reshape(N), srcp, dstp)
    part2 = _scat_kernel(h1, w2, src2d, dst2d, zerosH)

    bs = jnp.asarray(batch_size, jnp.int32).reshape(1)
    z = _tc_post(part2, h1, x0, ws1,
                 c1['bn_g'], c1['bn_b'], c1['bn_rm'], c1['bn_rv'],
                 params, bs)
    return z.reshape(1024)


# speedup vs baseline: 12.7516x; 1.4529x over previous
import functools

import jax
import jax.numpy as jnp
from jax import lax
from jax.experimental import pallas as pl
from jax.experimental.pallas import tpu as pltpu
import jax.experimental.pallas.tpu_sc as plsc

N = 10000
E = 320000
H = 128
EPS = 0.1
BN_EPS = 1e-5

NC = 2
NS = 16
NW = NC * NS
COPY_TILES = 10
NPC = N // COPY_TILES

G2 = 128
RPT2 = 80
E2 = NW * RPT2 * G2
EPT2 = E2 // NW
BLK = 8
NBLK = RPT2 // BLK
NPAD = N + 16

_mesh = plsc.VectorSubcoreMesh(core_axis_name="c", subcore_axis_name="s")


def _wid(c, s):
    return c * NS + s


HALVES = 2
NH = N // HALVES


def _deg_body(dstf, out, dst_v, hist_v, tot_v):
    c = lax.axis_index("c")
    s = lax.axis_index("s")
    t = _wid(c, s)
    pltpu.sync_copy(dstf.at[pl.ds(t * EPT2, EPT2)], dst_v)
    lane = jnp.arange(16, dtype=jnp.int32)
    zeros = jnp.zeros((16,), jnp.float32)
    ones = jnp.ones((16,), jnp.float32)

    for p in range(HALVES):
        def zstep(i, carry):
            hist_v[pl.ds(i * 16, 16)] = zeros
            return carry

        lax.fori_loop(0, NH, zstep, 0)

        base = p * NH

        def hstep(g, carry):
            dv = dst_v[pl.ds(g * 16, 16)]
            local = dv - base
            m = (local >= 0) & (local < NH)
            lidx = jnp.clip(local, 0, NH - 1) * 16 + lane
            plsc.addupdate_scatter(hist_v, [lidx], ones, mask=m)
            return carry

        lax.fori_loop(0, EPT2 // 16, hstep, 0)

        def rstep(r, carry):
            row16 = jnp.full((16,), r * 16, jnp.int32) + lane
            acc = zeros
            for i in range(16):
                acc = acc + plsc.load_gather(
                    hist_v, [row16 * 16 + jnp.full((16,), i, jnp.int32)])
            tot_v[pl.ds(r * 16, 16)] = acc
            return carry

        lax.fori_loop(0, NH // 16, rstep, 0)
        pltpu.sync_copy(tot_v, out.at[pl.ds(t * N + base, NH)])


_sc_params = pltpu.CompilerParams(needs_layout_passes=False)

_deg_kernel = pl.kernel(
    _deg_body,
    out_type=jax.ShapeDtypeStruct((NW * N,), jnp.float32),
    mesh=_mesh,
    compiler_params=_sc_params,
    scratch_types=[
        pltpu.VMEM((EPT2,), jnp.int32),
        pltpu.VMEM((NH * 16,), jnp.float32),
        pltpu.VMEM((NH,), jnp.float32),
    ],
)


def _w_body(hl, hr, dis, srcf, dstf, wout, hl_v, hr_v, dis_v, srcf_v,
            dstf_v, w_v):
    c = lax.axis_index("c")
    s = lax.axis_index("s")
    t = _wid(c, s)
    base = t * EPT2
    pltpu.sync_copy(hl, hl_v)
    pltpu.sync_copy(hr, hr_v)
    pltpu.sync_copy(dis, dis_v)
    pltpu.sync_copy(srcf.at[pl.ds(base, EPT2)], srcf_v)
    pltpu.sync_copy(dstf.at[pl.ds(base, EPT2)], dstf_v)
    lane = jnp.arange(16, dtype=jnp.int32)

    def wstep(g, carry):
        sl = pl.ds(g * 16, 16)
        sv = srcf_v[sl]
        dv = dstf_v[sl]
        a = plsc.load_gather(hl_v, [sv]) + plsc.load_gather(hr_v, [dv])
        ex = jnp.exp(-2.0 * jnp.abs(a))
        th = jnp.sign(a) * ((1.0 - ex) / (1.0 + ex))
        w = th * plsc.load_gather(dis_v, [sv]) * plsc.load_gather(dis_v, [dv])
        eidx = jnp.full((16,), base + g * 16, jnp.int32) + lane
        w_v[sl] = jnp.where(eidx < E, w, 0.0)
        return carry

    lax.fori_loop(0, EPT2 // 16, wstep, 0)
    pltpu.sync_copy(w_v, wout.at[pl.ds(base, EPT2)])


_w_kernel = pl.kernel(
    _w_body,
    out_type=jax.ShapeDtypeStruct((E2,), jnp.float32),
    mesh=_mesh,
    compiler_params=_sc_params,
    scratch_types=[
        pltpu.VMEM((N,), jnp.float32),
        pltpu.VMEM((N,), jnp.float32),
        pltpu.VMEM((N,), jnp.float32),
        pltpu.VMEM((EPT2,), jnp.int32),
        pltpu.VMEM((EPT2,), jnp.int32),
        pltpu.VMEM((EPT2,), jnp.float32),
    ],
)


def _scat_body(h, w, src2d, dst2d, zerosH, out,
               src_v, dst_v, w_v, rows_a, rows_b, acc_sh,
               gsem_a, gsem_b):
    c = lax.axis_index("c")
    s = lax.axis_index("s")
    t = _wid(c, s)
    rows = (rows_a, rows_b)
    gsem = (gsem_a, gsem_b)
    lane = jnp.arange(16, dtype=jnp.int32)

    @pl.when(s == 0)
    def _():
        pltpu.sync_copy(zerosH, acc_sh)

    plsc.subcore_barrier()

    def _scale(buf, w_base):
        def sgrp(g, carry):
            r0 = g * 16
            for rr in range(16):
                widx = jnp.full((16,), w_base + r0 + rr, jnp.int32)
                wv = plsc.load_gather(w_v, [widx])
                rfull = jnp.full((16,), r0 + rr, jnp.int32)
                for k in range(H // 16):
                    col = jnp.full((16,), k * 16, jnp.int32) + lane
                    v = plsc.load_gather(buf, [rfull, col])
                    plsc.store_scatter(buf, [rfull, col], v * wv)
            return carry

        lax.fori_loop(0, G2 // 16, sgrp, 0)

    def blk(b, carry):
        row0 = t * RPT2 + b * BLK
        pltpu.sync_copy(src2d.at[pl.ds(row0, BLK)], src_v)
        pltpu.sync_copy(dst2d.at[pl.ds(row0, BLK)], dst_v)
        pltpu.sync_copy(w.at[pl.ds(t * EPT2 + b * (BLK * G2), BLK * G2)], w_v)
        pltpu.async_copy(h.at[src_v.at[0]], rows[0], gsem[0])
        for jj in range(BLK):
            X = jj % 2
            Y = 1 - X
            pltpu.make_async_copy(h.at[src_v.at[jj]], rows[X],
                                  gsem[X]).wait()
            if jj < BLK - 1:
                pltpu.async_copy(h.at[src_v.at[jj + 1]], rows[Y], gsem[Y])
            _scale(rows[X], jj * G2)
            pltpu.sync_copy(rows[X], acc_sh.at[dst_v.at[jj]], add=True)
        return carry

    lax.fori_loop(0, NBLK, blk, 0)
    plsc.subcore_barrier()

    @pl.when(s < COPY_TILES)
    def _():
        pltpu.sync_copy(acc_sh.at[pl.ds(s * NPC, NPC)],
                        out.at[pl.ds(c * N + s * NPC, NPC)])


_scat_kernel = pl.kernel(
    _scat_body,
    out_type=jax.ShapeDtypeStruct((2 * N, H), jnp.float32),
    mesh=_mesh,
    compiler_params=_sc_params,
    scratch_types=[
        pltpu.VMEM((BLK, G2), jnp.int32),
        pltpu.VMEM((BLK, G2), jnp.int32),
        pltpu.VMEM((BLK * G2,), jnp.float32),
        pltpu.VMEM((G2, H), jnp.float32),
        pltpu.VMEM((G2, H), jnp.float32),
        pltpu.VMEM_SHARED((NPAD, H), jnp.float32),
        pltpu.SemaphoreType.DMA,
        pltpu.SemaphoreType.DMA,
    ],
)


def _bn(z, g, b, rm, rv):
    return (z - rm) / jnp.sqrt(rv + BN_EPS) * g + b


def _pre_body(x_ref, w_ref, b_ref, al_ref, ar_ref, dp_ref,
              x0_ref, hl_ref, hr_ref, dis_ref, ws_ref):
    x0 = jnp.maximum(
        jnp.dot(x_ref[...], w_ref[...], preferred_element_type=jnp.float32)
        + b_ref[...][None, :], 0.0)
    x0_ref[...] = x0
    hl = jnp.dot(x0, al_ref[...], preferred_element_type=jnp.float32)
    hr = jnp.dot(x0, ar_ref[...], preferred_element_type=jnp.float32)
    hl_ref[...] = hl
    hr_ref[...] = hr
    deg = jnp.sum(dp_ref[...], axis=0)[:, None] + 1.0
    dis = lax.rsqrt(deg)
    dis_ref[...] = dis
    ws_ref[...] = jnp.tanh(hl + hr) * dis * dis


def _tc_pre(x, w, b, al, ar, degpart):
    return pl.pallas_call(
        _pre_body,
        out_shape=(
            jax.ShapeDtypeStruct((N, H), jnp.float32),
            jax.ShapeDtypeStruct((N, 1), jnp.float32),
            jax.ShapeDtypeStruct((N, 1), jnp.float32),
            jax.ShapeDtypeStruct((N, 1), jnp.float32),
            jax.ShapeDtypeStruct((N, 1), jnp.float32),
        ),
    )(x, w, b, al, ar, degpart)


def _mid_body(part_ref, hprev_ref, x0_ref, ws_ref, dis_ref,
              g_ref, b_ref, rm_ref, rv_ref, al_ref, ar_ref,
              h1_ref, hl_ref, hr_ref, ws1_ref):
    part = part_ref[...]
    out = (part[0:N] + part[N:2 * N]
           + ws_ref[...] * hprev_ref[...] + EPS * x0_ref[...])
    h1 = _bn(jnp.maximum(out, 0.0), g_ref[...][None, :], b_ref[...][None, :],
             rm_ref[...][None, :], rv_ref[...][None, :])
    h1_ref[...] = h1
    hl = jnp.dot(h1, al_ref[...], preferred_element_type=jnp.float32)
    hr = jnp.dot(h1, ar_ref[...], preferred_element_type=jnp.float32)
    hl_ref[...] = hl
    hr_ref[...] = hr
    dis = dis_ref[...]
    ws1_ref[...] = jnp.tanh(hl + hr) * dis * dis


def _tc_mid(part, hprev, x0, ws, dis, g, b, rm, rv, al, ar):
    return pl.pallas_call(
        _mid_body,
        out_shape=(
            jax.ShapeDtypeStruct((N, H), jnp.float32),
            jax.ShapeDtypeStruct((N, 1), jnp.float32),
            jax.ShapeDtypeStruct((N, 1), jnp.float32),
            jax.ShapeDtypeStruct((N, 1), jnp.float32),
        ),
    )(part, hprev, x0, ws, dis, g, b, rm, rv, al, ar)


def _post_body(part_ref, hprev_ref, x0_ref, ws_ref,
               g_ref, b_ref, rm_ref, rv_ref,
               w1_ref, b1_ref, g1_ref, bb1_ref, rm1_ref, rv1_ref,
               w2_ref, b2_ref, g2_ref, bb2_ref, rm2_ref, rv2_ref,
               w3_ref, b3_ref, bs_ref, z_ref, h2_ref):
    part = part_ref[...]
    out = (part[0:N] + part[N:2 * N]
           + ws_ref[...] * hprev_ref[...] + EPS * x0_ref[...])
    h2_ref[...] = _bn(jnp.maximum(out, 0.0), g_ref[...][None, :],
                      b_ref[...][None, :], rm_ref[...][None, :],
                      rv_ref[...][None, :])
    start = bs_ref[0] - 1024
    z = h2_ref[pl.ds(start, 1024), :]
    z = jnp.maximum(_bn(
        jnp.dot(z, w1_ref[...], preferred_element_type=jnp.float32)
        + b1_ref[...][None, :],
        g1_ref[...][None, :], bb1_ref[...][None, :],
        rm1_ref[...][None, :], rv1_ref[...][None, :]), 0.0)
    z = jnp.maximum(_bn(
        jnp.dot(z, w2_ref[...], preferred_element_type=jnp.float32)
        + b2_ref[...][None, :],
        g2_ref[...][None, :], bb2_ref[...][None, :],
        rm2_ref[...][None, :], rv2_ref[...][None, :]), 0.0)
    z_ref[...] = (jnp.dot(z, w3_ref[...], preferred_element_type=jnp.float32)
                  + b3_ref[...][None, :])


def _tc_post(part, hprev, x0, ws, g, b, rm, rv, p, bs):
    in_specs = [pl.BlockSpec(memory_space=pltpu.VMEM) for _ in range(22)]
    in_specs.append(pl.BlockSpec(memory_space=pltpu.SMEM))
    return pl.pallas_call(
        _post_body,
        out_shape=jax.ShapeDtypeStruct((1024, 1), jnp.float32),
        in_specs=in_specs,
        out_specs=pl.BlockSpec(memory_space=pltpu.VMEM),
        scratch_shapes=[pltpu.VMEM((N, H), jnp.float32)],
    )(part, hprev, x0, ws, g, b, rm, rv,
      p['W1'], p['b1'], p['bn1_g'], p['bn1_b'], p['bn1_rm'], p['bn1_rv'],
      p['W2'], p['b2'], p['bn2_g'], p['bn2_b'], p['bn2_rm'], p['bn2_rv'],
      p['W3'], p['b3'], bs)


def kernel(x, edge_index, batch_size, params):
    src = edge_index[0]
    dst = edge_index[1]
    srcp = jnp.concatenate([src, jnp.zeros((E2 - E,), jnp.int32)])
    dstp = jnp.concatenate([dst, jnp.full((E2 - E,), N, jnp.int32)])
    src2d = srcp.reshape(NW * RPT2, G2)
    dst2d = dstp.reshape(NW * RPT2, G2)
    zerosH = jnp.zeros((NPAD, H), jnp.float32)

    degpart = _deg_kernel(dstp).reshape(NW, N)

    c0, c1 = params['convs'][0], params['convs'][1]
    al0 = c0['a_l'].reshape(H, 1)
    ar0 = c0['a_r'].reshape(H, 1)
    al1 = c1['a_l'].reshape(H, 1)
    ar1 = c1['a_r'].reshape(H, 1)

    x0, hl0, hr0, dis, ws0 = _tc_pre(x, params['W_in'], params['b_in'],
                                     al0, ar0, degpart)

    w1 = _w_kernel(hl0.reshape(N), hr0.reshape(N), dis.reshape(N), srcp, dstp)
    part1 = _scat_kernel(x0, w1, src2d, dst2d, zerosH)

    h1, hl1, hr1, ws1 = _tc_mid(part1, x0, x0, ws0, dis,
                                c0['bn_g'], c0['bn_b'], c0['bn_rm'],
                                c0['bn_rv'], al1, ar1)

    w2 = _w_kernel(hl1.reshape(N), hr1.reshape(N), dis.reshape(N), srcp, dstp)
    part2 = _scat_kernel(h1, w2, src2d, dst2d, zerosH)

    bs = jnp.asarray(batch_size, jnp.int32).reshape(1)
    z = _tc_post(part2, h1, x0, ws1,
                 c1['bn_g'], c1['bn_b'], c1['bn_rm'], c1['bn_rv'],
                 params, bs)
    return z.reshape(1024)


# speedup vs baseline: 12.7548x; 1.0002x over previous
import functools

import jax
import jax.numpy as jnp
from jax import lax
from jax.experimental import pallas as pl
from jax.experimental.pallas import tpu as pltpu
import jax.experimental.pallas.tpu_sc as plsc

N = 10000
E = 320000
H = 128
EPS = 0.1
BN_EPS = 1e-5

NC = 2
NS = 16
NW = NC * NS
COPY_TILES = 10
NPC = N // COPY_TILES

G2 = 128
RPT2 = 80
E2 = NW * RPT2 * G2
EPT2 = E2 // NW
BLK = 8
NBLK = RPT2 // BLK
NPAD = N + 16

_mesh = plsc.VectorSubcoreMesh(core_axis_name="c", subcore_axis_name="s")


def _wid(c, s):
    return c * NS + s


HALVES = 2
NH = N // HALVES


def _deg_body(dstf, out, dst_v, hist_v, tot_v):
    c = lax.axis_index("c")
    s = lax.axis_index("s")
    t = _wid(c, s)
    pltpu.sync_copy(dstf.at[pl.ds(t * EPT2, EPT2)], dst_v)
    lane = jnp.arange(16, dtype=jnp.int32)
    zeros = jnp.zeros((16,), jnp.float32)
    ones = jnp.ones((16,), jnp.float32)

    for p in range(HALVES):
        def zstep(i, carry):
            hist_v[pl.ds(i * 16, 16)] = zeros
            return carry

        lax.fori_loop(0, NH, zstep, 0)

        base = p * NH

        def hstep(g, carry):
            dv = dst_v[pl.ds(g * 16, 16)]
            local = dv - base
            m = (local >= 0) & (local < NH)
            lidx = jnp.clip(local, 0, NH - 1) * 16 + lane
            plsc.addupdate_scatter(hist_v, [lidx], ones, mask=m)
            return carry

        lax.fori_loop(0, EPT2 // 16, hstep, 0)

        def rstep(r, carry):
            row16 = jnp.full((16,), r * 16, jnp.int32) + lane
            acc = zeros
            for i in range(16):
                acc = acc + plsc.load_gather(
                    hist_v, [row16 * 16 + jnp.full((16,), i, jnp.int32)])
            tot_v[pl.ds(r * 16, 16)] = acc
            return carry

        lax.fori_loop(0, NH // 16, rstep, 0)
        pltpu.sync_copy(tot_v, out.at[pl.ds(t * N + base, NH)])


_sc_params = pltpu.CompilerParams(needs_layout_passes=False)

_deg_kernel = pl.kernel(
    _deg_body,
    out_type=jax.ShapeDtypeStruct((NW * N,), jnp.float32),
    mesh=_mesh,
    compiler_params=_sc_params,
    scratch_types=[
        pltpu.VMEM((EPT2,), jnp.int32),
        pltpu.VMEM((NH * 16,), jnp.float32),
        pltpu.VMEM((NH,), jnp.float32),
    ],
)


def _w_body(hl, hr, dis, srcf, dstf, wout, hl_v, hr_v, dis_v, srcf_v,
            dstf_v, w_v):
    c = lax.axis_index("c")
    s = lax.axis_index("s")
    t = _wid(c, s)
    base = t * EPT2
    pltpu.sync_copy(hl, hl_v)
    pltpu.sync_copy(hr, hr_v)
    pltpu.sync_copy(dis, dis_v)
    pltpu.sync_copy(srcf.at[pl.ds(base, EPT2)], srcf_v)
    pltpu.sync_copy(dstf.at[pl.ds(base, EPT2)], dstf_v)
    lane = jnp.arange(16, dtype=jnp.int32)

    def wstep(g, carry):
        sl = pl.ds(g * 16, 16)
        sv = srcf_v[sl]
        dv = dstf_v[sl]
        a = plsc.load_gather(hl_v, [sv]) + plsc.load_gather(hr_v, [dv])
        ex = jnp.exp(-2.0 * jnp.abs(a))
        th = jnp.sign(a) * ((1.0 - ex) / (1.0 + ex))
        w = th * plsc.load_gather(dis_v, [sv]) * plsc.load_gather(dis_v, [dv])
        eidx = jnp.full((16,), base + g * 16, jnp.int32) + lane
        w_v[sl] = jnp.where(eidx < E, w, 0.0)
        return carry

    lax.fori_loop(0, EPT2 // 16, wstep, 0)
    pltpu.sync_copy(w_v, wout.at[pl.ds(base, EPT2)])


_w_kernel = pl.kernel(
    _w_body,
    out_type=jax.ShapeDtypeStruct((E2,), jnp.float32),
    mesh=_mesh,
    compiler_params=_sc_params,
    scratch_types=[
        pltpu.VMEM((N,), jnp.float32),
        pltpu.VMEM((N,), jnp.float32),
        pltpu.VMEM((N,), jnp.float32),
        pltpu.VMEM((EPT2,), jnp.int32),
        pltpu.VMEM((EPT2,), jnp.int32),
        pltpu.VMEM((EPT2,), jnp.float32),
    ],
)


def _scat_body(h, w, src2d, dst2d, zerosH, out,
               src_v, dst_v, w_v, rows_a, rows_b, acc_sh,
               gsem_a, gsem_b, ssem_a, ssem_b):
    c = lax.axis_index("c")
    s = lax.axis_index("s")
    t = _wid(c, s)
    rows = (rows_a, rows_b)
    gsem = (gsem_a, gsem_b)
    ssem = (ssem_a, ssem_b)
    lane = jnp.arange(16, dtype=jnp.int32)
    cols = [jnp.full((16,), k * 16, jnp.int32) + lane for k in range(H // 16)]

    @pl.when(s == 0)
    def _():
        pltpu.sync_copy(zerosH, acc_sh)

    plsc.subcore_barrier()

    def _scale(buf, w_base):
        wb = jnp.full((16,), w_base, jnp.int32)

        def sgrp(g, carry):
            r0v = jnp.full((16,), g * 16, jnp.int32)
            for rr in range(16):
                rfull = r0v + rr
                wv = plsc.load_gather(w_v, [wb + rfull])
                for k in range(H // 16):
                    v = plsc.load_gather(buf, [rfull, cols[k]])
                    plsc.store_scatter(buf, [rfull, cols[k]], v * wv)
            return carry

        lax.fori_loop(0, G2 // 16, sgrp, 0)

    def _wait_scat(Y):
        pltpu.make_async_copy(rows[Y], acc_sh.at[dst_v.at[0]], ssem[Y]).wait()

    def blk(b, carry):
        @pl.when(b > 0)
        def _():
            _wait_scat(0)
            _wait_scat(1)

        row0 = t * RPT2 + b * BLK
        pltpu.sync_copy(src2d.at[pl.ds(row0, BLK)], src_v)
        pltpu.sync_copy(dst2d.at[pl.ds(row0, BLK)], dst_v)
        pltpu.sync_copy(w.at[pl.ds(t * EPT2 + b * (BLK * G2), BLK * G2)], w_v)
        pltpu.async_copy(h.at[src_v.at[0]], rows[0], gsem[0])
        for jj in range(BLK):
            X = jj % 2
            Y = 1 - X
            pltpu.make_async_copy(h.at[src_v.at[jj]], rows[X],
                                  gsem[X]).wait()
            if jj < BLK - 1:
                if jj >= 1:
                    _wait_scat(Y)
                pltpu.async_copy(h.at[src_v.at[jj + 1]], rows[Y], gsem[Y])
            _scale(rows[X], jj * G2)
            pltpu.async_copy(rows[X], acc_sh.at[dst_v.at[jj]], ssem[X],
                             add=True)
        return carry

    lax.fori_loop(0, NBLK, blk, 0)
    _wait_scat(0)
    _wait_scat(1)
    plsc.subcore_barrier()

    @pl.when(s < COPY_TILES)
    def _():
        pltpu.sync_copy(acc_sh.at[pl.ds(s * NPC, NPC)],
                        out.at[pl.ds(c * N + s * NPC, NPC)])


_scat_kernel = pl.kernel(
    _scat_body,
    out_type=jax.ShapeDtypeStruct((2 * N, H), jnp.float32),
    mesh=_mesh,
    compiler_params=_sc_params,
    scratch_types=[
        pltpu.VMEM((BLK, G2), jnp.int32),
        pltpu.VMEM((BLK, G2), jnp.int32),
        pltpu.VMEM((BLK * G2,), jnp.float32),
        pltpu.VMEM((G2, H), jnp.float32),
        pltpu.VMEM((G2, H), jnp.float32),
        pltpu.VMEM_SHARED((NPAD, H), jnp.float32),
        pltpu.SemaphoreType.DMA,
        pltpu.SemaphoreType.DMA,
        pltpu.SemaphoreType.DMA,
        pltpu.SemaphoreType.DMA,
    ],
)


def _bn(z, g, b, rm, rv):
    return (z - rm) / jnp.sqrt(rv + BN_EPS) * g + b


def _pre_body(x_ref, w_ref, b_ref, al_ref, ar_ref, dp_ref,
              x0_ref, hl_ref, hr_ref, dis_ref, ws_ref):
    x0 = jnp.maximum(
        jnp.dot(x_ref[...], w_ref[...], preferred_element_type=jnp.float32)
        + b_ref[...][None, :], 0.0)
    x0_ref[...] = x0
    hl = jnp.dot(x0, al_ref[...], preferred_element_type=jnp.float32)
    hr = jnp.dot(x0, ar_ref[...], preferred_element_type=jnp.float32)
    hl_ref[...] = hl
    hr_ref[...] = hr
    deg = jnp.sum(dp_ref[...], axis=0)[:, None] + 1.0
    dis = lax.rsqrt(deg)
    dis_ref[...] = dis
    ws_ref[...] = jnp.tanh(hl + hr) * dis * dis


def _tc_pre(x, w, b, al, ar, degpart):
    return pl.pallas_call(
        _pre_body,
        out_shape=(
            jax.ShapeDtypeStruct((N, H), jnp.float32),
            jax.ShapeDtypeStruct((N, 1), jnp.float32),
            jax.ShapeDtypeStruct((N, 1), jnp.float32),
            jax.ShapeDtypeStruct((N, 1), jnp.float32),
            jax.ShapeDtypeStruct((N, 1), jnp.float32),
        ),
    )(x, w, b, al, ar, degpart)


def _mid_body(part_ref, hprev_ref, x0_ref, ws_ref, dis_ref,
              g_ref, b_ref, rm_ref, rv_ref, al_ref, ar_ref,
              h1_ref, hl_ref, hr_ref, ws1_ref):
    part = part_ref[...]
    out = (part[0:N] + part[N:2 * N]
           + ws_ref[...] * hprev_ref[...] + EPS * x0_ref[...])
    h1 = _bn(jnp.maximum(out, 0.0), g_ref[...][None, :], b_ref[...][None, :],
             rm_ref[...][None, :], rv_ref[...][None, :])
    h1_ref[...] = h1
    hl = jnp.dot(h1, al_ref[...], preferred_element_type=jnp.float32)
    hr = jnp.dot(h1, ar_ref[...], preferred_element_type=jnp.float32)
    hl_ref[...] = hl
    hr_ref[...] = hr
    dis = dis_ref[...]
    ws1_ref[...] = jnp.tanh(hl + hr) * dis * dis


def _tc_mid(part, hprev, x0, ws, dis, g, b, rm, rv, al, ar):
    return pl.pallas_call(
        _mid_body,
        out_shape=(
            jax.ShapeDtypeStruct((N, H), jnp.float32),
            jax.ShapeDtypeStruct((N, 1), jnp.float32),
            jax.ShapeDtypeStruct((N, 1), jnp.float32),
            jax.ShapeDtypeStruct((N, 1), jnp.float32),
        ),
    )(part, hprev, x0, ws, dis, g, b, rm, rv, al, ar)


def _post_body(part_ref, hprev_ref, x0_ref, ws_ref,
               g_ref, b_ref, rm_ref, rv_ref,
               w1_ref, b1_ref, g1_ref, bb1_ref, rm1_ref, rv1_ref,
               w2_ref, b2_ref, g2_ref, bb2_ref, rm2_ref, rv2_ref,
               w3_ref, b3_ref, bs_ref, z_ref, h2_ref):
    part = part_ref[...]
    out = (part[0:N] + part[N:2 * N]
           + ws_ref[...] * hprev_ref[...] + EPS * x0_ref[...])
    h2_ref[...] = _bn(jnp.maximum(out, 0.0), g_ref[...][None, :],
                      b_ref[...][None, :], rm_ref[...][None, :],
                      rv_ref[...][None, :])
    start = bs_ref[0] - 1024
    z = h2_ref[pl.ds(start, 1024), :]
    z = jnp.maximum(_bn(
        jnp.dot(z, w1_ref[...], preferred_element_type=jnp.float32)
        + b1_ref[...][None, :],
        g1_ref[...][None, :], bb1_ref[...][None, :],
        rm1_ref[...][None, :], rv1_ref[...][None, :]), 0.0)
    z = jnp.maximum(_bn(
        jnp.dot(z, w2_ref[...], preferred_element_type=jnp.float32)
        + b2_ref[...][None, :],
        g2_ref[...][None, :], bb2_ref[...][None, :],
        rm2_ref[...][None, :], rv2_ref[...][None, :]), 0.0)
    z_ref[...] = (jnp.dot(z, w3_ref[...], preferred_element_type=jnp.float32)
                  + b3_ref[...][None, :])


def _tc_post(part, hprev, x0, ws, g, b, rm, rv, p, bs):
    in_specs = [pl.BlockSpec(memory_space=pltpu.VMEM) for _ in range(22)]
    in_specs.append(pl.BlockSpec(memory_space=pltpu.SMEM))
    return pl.pallas_call(
        _post_body,
        out_shape=jax.ShapeDtypeStruct((1024, 1), jnp.float32),
        in_specs=in_specs,
        out_specs=pl.BlockSpec(memory_space=pltpu.VMEM),
        scratch_shapes=[pltpu.VMEM((N, H), jnp.float32)],
    )(part, hprev, x0, ws, g, b, rm, rv,
      p['W1'], p['b1'], p['bn1_g'], p['bn1_b'], p['bn1_rm'], p['bn1_rv'],
      p['W2'], p['b2'], p['bn2_g'], p['bn2_b'], p['bn2_rm'], p['bn2_rv'],
      p['W3'], p['b3'], bs)


def kernel(x, edge_index, batch_size, params):
    src = edge_index[0]
    dst = edge_index[1]
    srcp = jnp.concatenate([src, jnp.zeros((E2 - E,), jnp.int32)])
    dstp = jnp.concatenate([dst, jnp.full((E2 - E,), N, jnp.int32)])
    src2d = srcp.reshape(NW * RPT2, G2)
    dst2d = dstp.reshape(NW * RPT2, G2)
    zerosH = jnp.zeros((NPAD, H), jnp.float32)

    degpart = _deg_kernel(dstp).reshape(NW, N)

    c0, c1 = params['convs'][0], params['convs'][1]
    al0 = c0['a_l'].reshape(H, 1)
    ar0 = c0['a_r'].reshape(H, 1)
    al1 = c1['a_l'].reshape(H, 1)
    ar1 = c1['a_r'].reshape(H, 1)

    x0, hl0, hr0, dis, ws0 = _tc_pre(x, params['W_in'], params['b_in'],
                                     al0, ar0, degpart)

    w1 = _w_kernel(hl0.reshape(N), hr0.reshape(N), dis.reshape(N), srcp, dstp)
    part1 = _scat_kernel(x0, w1, src2d, dst2d, zerosH)

    h1, hl1, hr1, ws1 = _tc_mid(part1, x0, x0, ws0, dis,
                                c0['bn_g'], c0['bn_b'], c0['bn_rm'],
                                c0['bn_rv'], al1, ar1)

    w2 = _w_kernel(hl1.reshape(N), hr1.reshape(N), dis.reshape(N), srcp, dstp)
    part2 = _scat_kernel(h1, w2, src2d, dst2d, zerosH)

    bs = jnp.asarray(batch_size, jnp.int32).reshape(1)
    z = _tc_post(part2, h1, x0, ws1,
                 c1['bn_g'], c1['bn_b'], c1['bn_rm'], c1['bn_rv'],
                 params, bs)
    return z.reshape(1024)


# speedup vs baseline: 13.9954x; 1.0973x over previous
import functools

import jax
import jax.numpy as jnp
from jax import lax
from jax.experimental import pallas as pl
from jax.experimental.pallas import tpu as pltpu
import jax.experimental.pallas.tpu_sc as plsc

N = 10000
E = 320000
H = 128
EPS = 0.1
BN_EPS = 1e-5

NC = 2
NS = 16
NW = NC * NS
COPY_TILES = 10
NPC = N // COPY_TILES

G2 = 128
RPT2 = 80
E2 = NW * RPT2 * G2
EPT2 = E2 // NW
BLK = 8
NBLK = RPT2 // BLK
NPAD = N + 16

_mesh = plsc.VectorSubcoreMesh(core_axis_name="c", subcore_axis_name="s")


def _wid(c, s):
    return c * NS + s


HALVES = 2
NH = N // HALVES


def _deg_body(dstf, out, dst_v, hist_v, tot_v):
    c = lax.axis_index("c")
    s = lax.axis_index("s")
    t = _wid(c, s)
    pltpu.sync_copy(dstf.at[pl.ds(t * EPT2, EPT2)], dst_v)
    lane = jnp.arange(16, dtype=jnp.int32)
    zeros = jnp.zeros((16,), jnp.float32)
    ones = jnp.ones((16,), jnp.float32)

    for p in range(HALVES):
        def zstep(i, carry):
            for u in range(10):
                hist_v[pl.ds(i * 160 + u * 16, 16)] = zeros
            return carry

        lax.fori_loop(0, (NH * 16) // 160, zstep, 0)

        base = p * NH

        def hstep(g, carry):
            dv = dst_v[pl.ds(g * 16, 16)]
            local = dv - base
            m = (local >= 0) & (local < NH)
            lidx = jnp.clip(local, 0, NH - 1) * 16 + lane
            plsc.addupdate_scatter(hist_v, [lidx], ones, mask=m)
            return carry

        lax.fori_loop(0, EPT2 // 16, hstep, 0)

        def rstep(r, carry):
            row16 = jnp.full((16,), r * 16, jnp.int32) + lane
            acc = zeros
            for i in range(16):
                acc = acc + plsc.load_gather(
                    hist_v, [row16 * 16 + jnp.full((16,), i, jnp.int32)])
            tot_v[pl.ds(r * 16, 16)] = acc
            return carry

        lax.fori_loop(0, NH // 16, rstep, 0)
        pltpu.sync_copy(tot_v, out.at[pl.ds(t * N + base, NH)])


_sc_params = pltpu.CompilerParams(needs_layout_passes=False)

_deg_kernel = pl.kernel(
    _deg_body,
    out_type=jax.ShapeDtypeStruct((NW * N,), jnp.float32),
    mesh=_mesh,
    compiler_params=_sc_params,
    scratch_types=[
        pltpu.VMEM((EPT2,), jnp.int32),
        pltpu.VMEM((NH * 16,), jnp.float32),
        pltpu.VMEM((NH,), jnp.float32),
    ],
)


def _w_body(hl, hr, dis, srcf, dstf, wout, hl_v, hr_v, dis_v, srcf_v,
            dstf_v, w_v):
    c = lax.axis_index("c")
    s = lax.axis_index("s")
    t = _wid(c, s)
    base = t * EPT2
    pltpu.sync_copy(hl, hl_v)
    pltpu.sync_copy(hr, hr_v)
    pltpu.sync_copy(dis, dis_v)
    pltpu.sync_copy(srcf.at[pl.ds(base, EPT2)], srcf_v)
    pltpu.sync_copy(dstf.at[pl.ds(base, EPT2)], dstf_v)
    lane = jnp.arange(16, dtype=jnp.int32)

    def wstep(g, carry):
        sl = pl.ds(g * 16, 16)
        sv = srcf_v[sl]
        dv = dstf_v[sl]
        a = plsc.load_gather(hl_v, [sv]) + plsc.load_gather(hr_v, [dv])
        ex = jnp.exp(-2.0 * jnp.abs(a))
        th = jnp.sign(a) * ((1.0 - ex) / (1.0 + ex))
        w = th * plsc.load_gather(dis_v, [sv]) * plsc.load_gather(dis_v, [dv])
        eidx = jnp.full((16,), base + g * 16, jnp.int32) + lane
        w_v[sl] = jnp.where(eidx < E, w, 0.0)
        return carry

    lax.fori_loop(0, EPT2 // 16, wstep, 0)
    pltpu.sync_copy(w_v, wout.at[pl.ds(base, EPT2)])


_w_kernel = pl.kernel(
    _w_body,
    out_type=jax.ShapeDtypeStruct((E2,), jnp.float32),
    mesh=_mesh,
    compiler_params=_sc_params,
    scratch_types=[
        pltpu.VMEM((N,), jnp.float32),
        pltpu.VMEM((N,), jnp.float32),
        pltpu.VMEM((N,), jnp.float32),
        pltpu.VMEM((EPT2,), jnp.int32),
        pltpu.VMEM((EPT2,), jnp.int32),
        pltpu.VMEM((EPT2,), jnp.float32),
    ],
)


def _scat_body(h, w, src2d, dst2d, zerosH, out,
               src_v, dst_v, w_v, rows_a, rows_b, acc_sh,
               gsem_a, gsem_b, ssem_a, ssem_b):
    c = lax.axis_index("c")
    s = lax.axis_index("s")
    t = _wid(c, s)
    rows = (rows_a, rows_b)
    gsem = (gsem_a, gsem_b)
    ssem = (ssem_a, ssem_b)
    lane = jnp.arange(16, dtype=jnp.int32)
    cols = [jnp.full((16,), k * 16, jnp.int32) + lane for k in range(H // 16)]

    @pl.when(s == 0)
    def _():
        pltpu.sync_copy(zerosH, acc_sh)

    plsc.subcore_barrier()

    def _scale(buf, w_base):
        wb = jnp.full((16,), w_base, jnp.int32)

        def sgrp(g, carry):
            r0v = jnp.full((16,), g * 16, jnp.int32)
            for rr in range(16):
                rfull = r0v + rr
                wv = plsc.load_gather(w_v, [wb + rfull])
                for k in range(H // 16):
                    v = plsc.load_gather(buf, [rfull, cols[k]])
                    plsc.store_scatter(buf, [rfull, cols[k]], v * wv)
            return carry

        lax.fori_loop(0, G2 // 16, sgrp, 0)

    def _wait_scat(Y):
        pltpu.make_async_copy(rows[Y], acc_sh.at[dst_v.at[0]], ssem[Y]).wait()

    pltpu.sync_copy(src2d.at[pl.ds(t * RPT2, BLK)], src_v)
    pltpu.async_copy(h.at[src_v.at[0]], rows[0], gsem[0])

    def blk(b, carry):
        @pl.when(b > 0)
        def _():
            _wait_scat(1)

        row0 = t * RPT2 + b * BLK
        pltpu.sync_copy(dst2d.at[pl.ds(row0, BLK)], dst_v)
        pltpu.sync_copy(w.at[pl.ds(t * EPT2 + b * (BLK * G2), BLK * G2)], w_v)
        for jj in range(BLK):
            X = jj % 2
            Y = 1 - X
            pltpu.make_async_copy(h.at[src_v.at[jj]], rows[X],
                                  gsem[X]).wait()
            if jj < BLK - 1:
                if jj >= 1:
                    _wait_scat(Y)
                pltpu.async_copy(h.at[src_v.at[jj + 1]], rows[Y], gsem[Y])
            else:
                @pl.when(b < NBLK - 1)
                def _():
                    _wait_scat(0)
                    pltpu.sync_copy(src2d.at[pl.ds(row0 + BLK, BLK)], src_v)
                    pltpu.async_copy(h.at[src_v.at[0]], rows[0], gsem[0])
            _scale(rows[X], jj * G2)
            pltpu.async_copy(rows[X], acc_sh.at[dst_v.at[jj]], ssem[X],
                             add=True)
        return carry

    lax.fori_loop(0, NBLK, blk, 0)
    _wait_scat(0)
    _wait_scat(1)
    plsc.subcore_barrier()

    @pl.when(s < COPY_TILES)
    def _():
        pltpu.sync_copy(acc_sh.at[pl.ds(s * NPC, NPC)],
                        out.at[pl.ds(c * N + s * NPC, NPC)])


_scat_kernel = pl.kernel(
    _scat_body,
    out_type=jax.ShapeDtypeStruct((2 * N, H), jnp.float32),
    mesh=_mesh,
    compiler_params=_sc_params,
    scratch_types=[
        pltpu.VMEM((BLK, G2), jnp.int32),
        pltpu.VMEM((BLK, G2), jnp.int32),
        pltpu.VMEM((BLK * G2,), jnp.float32),
        pltpu.VMEM((G2, H), jnp.float32),
        pltpu.VMEM((G2, H), jnp.float32),
        pltpu.VMEM_SHARED((NPAD, H), jnp.float32),
        pltpu.SemaphoreType.DMA,
        pltpu.SemaphoreType.DMA,
        pltpu.SemaphoreType.DMA,
        pltpu.SemaphoreType.DMA,
    ],
)


def _bn(z, g, b, rm, rv):
    return (z - rm) / jnp.sqrt(rv + BN_EPS) * g + b


def _pre_body(x_ref, w_ref, b_ref, al_ref, ar_ref, dp_ref,
              x0_ref, hl_ref, hr_ref, dis_ref, ws_ref):
    x0 = jnp.maximum(
        jnp.dot(x_ref[...], w_ref[...], preferred_element_type=jnp.float32)
        + b_ref[...][None, :], 0.0)
    x0_ref[...] = x0
    hl = jnp.dot(x0, al_ref[...], preferred_element_type=jnp.float32)
    hr = jnp.dot(x0, ar_ref[...], preferred_element_type=jnp.float32)
    hl_ref[...] = hl
    hr_ref[...] = hr
    deg = jnp.sum(dp_ref[...], axis=0)[:, None] + 1.0
    dis = lax.rsqrt(deg)
    dis_ref[...] = dis
    ws_ref[...] = jnp.tanh(hl + hr) * dis * dis


def _tc_pre(x, w, b, al, ar, degpart):
    return pl.pallas_call(
        _pre_body,
        out_shape=(
            jax.ShapeDtypeStruct((N, H), jnp.float32),
            jax.ShapeDtypeStruct((N, 1), jnp.float32),
            jax.ShapeDtypeStruct((N, 1), jnp.float32),
            jax.ShapeDtypeStruct((N, 1), jnp.float32),
            jax.ShapeDtypeStruct((N, 1), jnp.float32),
        ),
    )(x, w, b, al, ar, degpart)


def _mid_body(part_ref, hprev_ref, x0_ref, ws_ref, dis_ref,
              g_ref, b_ref, rm_ref, rv_ref, al_ref, ar_ref,
              h1_ref, hl_ref, hr_ref, ws1_ref):
    part = part_ref[...]
    out = (part[0:N] + part[N:2 * N]
           + ws_ref[...] * hprev_ref[...] + EPS * x0_ref[...])
    h1 = _bn(jnp.maximum(out, 0.0), g_ref[...][None, :], b_ref[...][None, :],
             rm_ref[...][None, :], rv_ref[...][None, :])
    h1_ref[...] = h1
    hl = jnp.dot(h1, al_ref[...], preferred_element_type=jnp.float32)
    hr = jnp.dot(h1, ar_ref[...], preferred_element_type=jnp.float32)
    hl_ref[...] = hl
    hr_ref[...] = hr
    dis = dis_ref[...]
    ws1_ref[...] = jnp.tanh(hl + hr) * dis * dis


def _tc_mid(part, hprev, x0, ws, dis, g, b, rm, rv, al, ar):
    return pl.pallas_call(
        _mid_body,
        out_shape=(
            jax.ShapeDtypeStruct((N, H), jnp.float32),
            jax.ShapeDtypeStruct((N, 1), jnp.float32),
            jax.ShapeDtypeStruct((N, 1), jnp.float32),
            jax.ShapeDtypeStruct((N, 1), jnp.float32),
        ),
    )(part, hprev, x0, ws, dis, g, b, rm, rv, al, ar)


def _post_body(part_ref, hprev_ref, x0_ref, ws_ref,
               g_ref, b_ref, rm_ref, rv_ref,
               w1_ref, b1_ref, g1_ref, bb1_ref, rm1_ref, rv1_ref,
               w2_ref, b2_ref, g2_ref, bb2_ref, rm2_ref, rv2_ref,
               w3_ref, b3_ref, bs_ref, z_ref, h2_ref):
    part = part_ref[...]
    out = (part[0:N] + part[N:2 * N]
           + ws_ref[...] * hprev_ref[...] + EPS * x0_ref[...])
    h2_ref[...] = _bn(jnp.maximum(out, 0.0), g_ref[...][None, :],
                      b_ref[...][None, :], rm_ref[...][None, :],
                      rv_ref[...][None, :])
    start = bs_ref[0] - 1024
    z = h2_ref[pl.ds(start, 1024), :]
    z = jnp.maximum(_bn(
        jnp.dot(z, w1_ref[...], preferred_element_type=jnp.float32)
        + b1_ref[...][None, :],
        g1_ref[...][None, :], bb1_ref[...][None, :],
        rm1_ref[...][None, :], rv1_ref[...][None, :]), 0.0)
    z = jnp.maximum(_bn(
        jnp.dot(z, w2_ref[...], preferred_element_type=jnp.float32)
        + b2_ref[...][None, :],
        g2_ref[...][None, :], bb2_ref[...][None, :],
        rm2_ref[...][None, :], rv2_ref[...][None, :]), 0.0)
    z_ref[...] = (jnp.dot(z, w3_ref[...], preferred_element_type=jnp.float32)
                  + b3_ref[...][None, :])


def _tc_post(part, hprev, x0, ws, g, b, rm, rv, p, bs):
    in_specs = [pl.BlockSpec(memory_space=pltpu.VMEM) for _ in range(22)]
    in_specs.append(pl.BlockSpec(memory_space=pltpu.SMEM))
    return pl.pallas_call(
        _post_body,
        out_shape=jax.ShapeDtypeStruct((1024, 1), jnp.float32),
        in_specs=in_specs,
        out_specs=pl.BlockSpec(memory_space=pltpu.VMEM),
        scratch_shapes=[pltpu.VMEM((N, H), jnp.float32)],
    )(part, hprev, x0, ws, g, b, rm, rv,
      p['W1'], p['b1'], p['bn1_g'], p['bn1_b'], p['bn1_rm'], p['bn1_rv'],
      p['W2'], p['b2'], p['bn2_g'], p['bn2_b'], p['bn2_rm'], p['bn2_rv'],
      p['W3'], p['b3'], bs)


def kernel(x, edge_index, batch_size, params):
    src = edge_index[0]
    dst = edge_index[1]
    srcp = jnp.concatenate([src, jnp.zeros((E2 - E,), jnp.int32)])
    dstp = jnp.concatenate([dst, jnp.full((E2 - E,), N, jnp.int32)])
    src2d = srcp.reshape(NW * RPT2, G2)
    dst2d = dstp.reshape(NW * RPT2, G2)
    zerosH = jnp.zeros((NPAD, H), jnp.float32)

    degpart = _deg_kernel(dstp).reshape(NW, N)

    c0, c1 = params['convs'][0], params['convs'][1]
    al0 = c0['a_l'].reshape(H, 1)
    ar0 = c0['a_r'].reshape(H, 1)
    al1 = c1['a_l'].reshape(H, 1)
    ar1 = c1['a_r'].reshape(H, 1)

    x0, hl0, hr0, dis, ws0 = _tc_pre(x, params['W_in'], params['b_in'],
                                     al0, ar0, degpart)

    w1 = _w_kernel(hl0.reshape(N), hr0.reshape(N), dis.reshape(N), srcp, dstp)
    part1 = _scat_kernel(x0, w1, src2d, dst2d, zerosH)

    h1, hl1, hr1, ws1 = _tc_mid(part1, x0, x0, ws0, dis,
                                c0['bn_g'], c0['bn_b'], c0['bn_rm'],
                                c0['bn_rv'], al1, ar1)

    w2 = _w_kernel(hl1.reshape(N), hr1.reshape(N), dis.reshape(N), srcp, dstp)
    part2 = _scat_kernel(h1, w2, src2d, dst2d, zerosH)

    bs = jnp.asarray(batch_size, jnp.int32).reshape(1)
    z = _tc_post(part2, h1, x0, ws1,
                 c1['bn_g'], c1['bn_b'], c1['bn_rm'], c1['bn_rv'],
                 params, bs)
    return z.reshape(1024)
